# Initial kernel scaffold; baseline (speedup 1.0000x reference)
import jax
import jax.numpy as jnp
from jax.experimental import pallas as pl


def kernel(score):
    raise NotImplementedError("write your pallas kernel here")


# speedup vs baseline: 5.4958x; 5.4958x over previous
import functools

import jax
import jax.numpy as jnp
from jax import lax
from jax.experimental import pallas as pl
from jax.experimental.pallas import tpu as pltpu
from jax.experimental.pallas import tpu_sc as plsc

B = 64
N = 8192
K = 256
L = 16
NBINS = 2048
NV = N // L
ROWS_PER_W = 2
_INT_MIN = -(2**31)

_mesh = plsc.VectorSubcoreMesh(core_axis_name="c", subcore_axis_name="s")


@functools.partial(
    pl.kernel,
    mesh=_mesh,
    compiler_params=pltpu.CompilerParams(needs_layout_passes=False),
    out_type=jax.ShapeDtypeStruct((B, N), jnp.float32),
    scratch_types=[
        pltpu.VMEM((N,), jnp.float32),
        pltpu.VMEM((N,), jnp.int32),
        pltpu.VMEM((N,), jnp.float32),
        pltpu.VMEM((NBINS,), jnp.int32),
        pltpu.VMEM((N + 2 * L,), jnp.int32),
        pltpu.VMEM((N + 2 * L,), jnp.int32),
    ],
)
def _topk_mask(score_hbm, out_hbm, rowf, keys, orow, hist, ckeys, cidx):
    wid = lax.axis_index("s") * 2 + lax.axis_index("c")
    lane = lax.iota(jnp.int32, L)
    zeros16 = jnp.zeros((L,), jnp.int32)
    ones16 = jnp.ones((L,), jnp.int32)

    for r in range(ROWS_PER_W):
        row = wid * ROWS_PER_W + r
        pltpu.sync_copy(score_hbm.at[row], rowf)

        def zh(i, _):
            hist[pl.ds(i * L, L)] = zeros16
            return 0

        lax.fori_loop(0, NBINS // L, zh, 0)

        def p1(i, _):
            f = rowf[pl.ds(i * L, L)]
            bits = lax.bitcast_convert_type(f, jnp.int32)
            s = jnp.where(bits >= 0, bits, _INT_MIN - bits)
            keys[pl.ds(i * L, L)] = s
            bin_ = (s >> 21) + 1024
            plsc.addupdate_scatter(hist, [bin_], ones16)
            return 0

        lax.fori_loop(0, NV, p1, 0)

        def wcond(c):
            return c[1] == 0

        def wbody(c):
            i, found, bb, above = c
            base = NBINS - L * (i + 1)
            v = hist[pl.ds(base, L)]
            rv = lax.rev(v, (0,))
            cs = above + plsc.cumsum(rv)
            crossed = cs >= K
            pc = plsc.all_reduce_population_count(crossed)
            pc0 = pc[0] if pc.ndim else pc
            jf = plsc.all_reduce_ffs(crossed)
            j = jf[0] if jf.ndim else jf
            above_b = jnp.sum(jnp.where(lane == j, cs - rv, 0))
            chunk_total = jnp.sum(v)
            hit = pc0 > 0
            return (
                i + 1,
                jnp.where(hit, 1, 0),
                jnp.where(hit, NBINS - 1 - L * i - j, bb),
                jnp.where(hit, above_b, above + chunk_total),
            )

        _, _, bb, above = lax.while_loop(wcond, wbody, (0, 0, 0, 0))
        k_rem = K - above

        def cp(i, off):
            s = keys[pl.ds(i * L, L)]
            m = ((s >> 21) + 1024) == bb
            plsc.store_compressed(ckeys.at[pl.ds(off, L)], s, mask=m)
            plsc.store_compressed(cidx.at[pl.ds(off, L)], lane + i * L, mask=m)
            pcv = plsc.all_reduce_population_count(m)
            return off + (pcv[0] if pcv.ndim else pcv)

        n = lax.fori_loop(0, NV, cp, jnp.int32(0))
        ckeys[pl.ds(n, L)] = jnp.full((L,), _INT_MIN, jnp.int32)
        nv = (n + L - 1) // L

        def rb(t, thr):
            cand = thr | (jnp.int32(1) << (20 - t))

            def cb(i, acc):
                return acc + jnp.where(ckeys[pl.ds(i * L, L)] >= cand, 1, 0)

            c = jnp.sum(lax.fori_loop(0, nv, cb, zeros16))
            return jnp.where(c >= k_rem, cand, thr)

        big_t = lax.fori_loop(0, 21, rb, (bb - 1024) << 21)

        def gb(i, acc):
            return acc + jnp.where(ckeys[pl.ds(i * L, L)] > big_t, 1, 0)

        gt = jnp.sum(lax.fori_loop(0, nv, gb, zeros16))
        t_need = k_rem - gt

        def ib(t, ipfx):
            cand = ipfx + (jnp.int32(1) << (12 - t))

            def cb2(i, acc):
                s = ckeys[pl.ds(i * L, L)]
                iv = cidx[pl.ds(i * L, L)]
                return acc + jnp.where((s == big_t) & (iv < cand), 1, 0)

            c = jnp.sum(lax.fori_loop(0, nv, cb2, zeros16))
            return jnp.where(c < t_need, cand, ipfx)

        cstar = lax.fori_loop(0, 13, ib, jnp.int32(0))

        def ob(i, _):
            s = keys[pl.ds(i * L, L)]
            iv = lane + i * L
            m = (s > big_t) | ((s == big_t) & (iv <= cstar))
            orow[pl.ds(i * L, L)] = jnp.where(m, 1.0, 0.0)
            return 0

        lax.fori_loop(0, NV, ob, 0)
        pltpu.sync_copy(orow, out_hbm.at[row])


def kernel(score):
    return _topk_mask(score)


# speedup vs baseline: 8.5489x; 1.5555x over previous
import functools

import jax
import jax.numpy as jnp
from jax import lax
from jax.experimental import pallas as pl
from jax.experimental.pallas import tpu as pltpu
from jax.experimental.pallas import tpu_sc as plsc

B = 64
N = 8192
K = 256
L = 16
NBINS = 2048
CBINS = 128
ROWS_PER_W = 2
_INT_MIN = -(2**31)

_mesh = plsc.VectorSubcoreMesh(core_axis_name="c", subcore_axis_name="s")


@functools.partial(
    pl.kernel,
    mesh=_mesh,
    compiler_params=pltpu.CompilerParams(needs_layout_passes=False),
    out_type=jax.ShapeDtypeStruct((B, N), jnp.float32),
    scratch_types=[
        pltpu.VMEM((N,), jnp.float32),
        pltpu.VMEM((N,), jnp.int32),
        pltpu.VMEM((N,), jnp.float32),
        pltpu.VMEM((NBINS,), jnp.int32),
        pltpu.VMEM((CBINS,), jnp.int32),
        pltpu.VMEM((CBINS,), jnp.int32),
        pltpu.VMEM((N + 2 * L,), jnp.int32),
        pltpu.VMEM((N + 2 * L,), jnp.int32),
        pltpu.VMEM((N + 2 * L,), jnp.int32),
        pltpu.VMEM((N + 2 * L,), jnp.int32),
    ],
)
def _topk_mask(
    score_hbm, out_hbm, rowf, keys, orow, hist, chist, hist2, ckeys, cidx,
    ckeys2, cidx2,
):
    wid = lax.axis_index("s") * 2 + lax.axis_index("c")
    lane = lax.iota(jnp.int32, L)
    zeros16 = jnp.zeros((L,), jnp.int32)
    ones16 = jnp.ones((L,), jnp.int32)

    def fine_step(href, base, kk, above_in):
        v = href[pl.ds(base, L)]
        rv = lax.rev(v, (0,))
        cs = above_in + plsc.cumsum(rv)
        jf = plsc.all_reduce_ffs(cs >= kk)
        j = jf[0] if jf.ndim else jf
        bin_ = base + (L - 1) - j
        above = jnp.sum(jnp.where(lane == j, cs - rv, 0))
        return bin_, above

    def walk(href, nbins, kk):
        def wcond(c):
            return c[1] == 0

        def wbody(c):
            i, found, j0, above, _cs, _rv = c
            v = href[pl.ds(nbins - L * (i + 1), L)]
            rv = lax.rev(v, (0,))
            cs = above + plsc.cumsum(rv)
            crossed = cs >= kk
            pc = plsc.all_reduce_population_count(crossed)
            pc0 = pc[0] if pc.ndim else pc
            jf = plsc.all_reduce_ffs(crossed)
            j = jf[0] if jf.ndim else jf
            hit = pc0 > 0
            return (
                i + 1,
                jnp.where(hit, 1, 0),
                jnp.where(hit, j, j0),
                jnp.where(hit, above, cs[L - 1]),
                jnp.where(hit, cs, _cs),
                jnp.where(hit, rv, _rv),
            )

        wi, _, wj, _, wcs, wrv = lax.while_loop(
            wcond, wbody, (0, 0, 0, 0, zeros16, zeros16)
        )
        bin_ = nbins - 1 - L * (wi - 1) - wj
        above = jnp.sum(jnp.where(lane == wj, wcs - wrv, 0))
        return bin_, above

    for r in range(ROWS_PER_W):
        row = wid * ROWS_PER_W + r
        pltpu.sync_copy(score_hbm.at[row], rowf)

        @plsc.parallel_loop(0, NBINS, L, unroll=8)
        def zh(i):
            hist[pl.ds(i, L)] = zeros16

        @plsc.parallel_loop(0, CBINS, L, unroll=8)
        def zc(i):
            chist[pl.ds(i, L)] = zeros16
            hist2[pl.ds(i, L)] = zeros16

        @plsc.parallel_loop(0, N, L, unroll=8)
        def p1(i):
            f = rowf[pl.ds(i, L)]
            bits = lax.bitcast_convert_type(f, jnp.int32)
            s = jnp.where(bits >= 0, bits, _INT_MIN - bits)
            keys[pl.ds(i, L)] = s
            plsc.addupdate_scatter(chist, [(s >> 25) + 64], ones16)
            plsc.addupdate_scatter(hist, [(s >> 21) + 1024], ones16)

        cb, above_c = walk(chist, CBINS, K)
        bb, above = fine_step(hist, cb * L, K, above_c)
        k_rem = K - above

        def cp(i, off):
            s = keys[pl.ds(i, L)]
            m = ((s >> 21) + 1024) == bb
            plsc.store_compressed(ckeys.at[pl.ds(off, L)], s, mask=m)
            plsc.store_compressed(cidx.at[pl.ds(off, L)], lane + i, mask=m)
            pcv = plsc.all_reduce_population_count(m)
            return off + (pcv[0] if pcv.ndim else pcv)

        n = plsc.parallel_loop(0, N, L, unroll=4, carry=jnp.int32(0))(cp)
        ckeys[pl.ds(n, L)] = jnp.full((L,), _INT_MIN, jnp.int32)
        nv = (n + L - 1) // L

        def h2(i):
            s = ckeys[pl.ds(i, L)]
            plsc.addupdate_scatter(
                hist2, [(s >> 14) & 127], ones16, mask=s != _INT_MIN
            )

        plsc.parallel_loop(0, nv * L, L)(h2)
        sb, above2 = walk(hist2, CBINS, k_rem)
        k_rem2 = k_rem - above2

        def cp2(i, off):
            s = ckeys[pl.ds(i, L)]
            iv = cidx[pl.ds(i, L)]
            m = (((s >> 14) & 127) == sb) & (s != _INT_MIN)
            plsc.store_compressed(ckeys2.at[pl.ds(off, L)], s, mask=m)
            plsc.store_compressed(cidx2.at[pl.ds(off, L)], iv, mask=m)
            pcv = plsc.all_reduce_population_count(m)
            return off + (pcv[0] if pcv.ndim else pcv)

        n2 = plsc.parallel_loop(0, nv * L, L, carry=jnp.int32(0))(cp2)
        ckeys2[pl.ds(n2, L)] = jnp.full((L,), _INT_MIN, jnp.int32)
        nv2 = (n2 + L - 1) // L

        def rb(t, thr):
            cand = thr | (jnp.int32(1) << (13 - t))

            def cb_(i, acc):
                return acc + jnp.where(ckeys2[pl.ds(i, L)] >= cand, 1, 0)

            c = jnp.sum(plsc.parallel_loop(0, nv2 * L, L, carry=zeros16)(cb_))
            return jnp.where(c >= k_rem2, cand, thr)

        base2 = ((bb - 1024) << 21) | (sb << 14)
        big_t = lax.fori_loop(0, 14, rb, base2)

        def gb(i, acc):
            return acc + jnp.where(ckeys2[pl.ds(i, L)] > big_t, 1, 0)

        gt = jnp.sum(plsc.parallel_loop(0, nv2 * L, L, carry=zeros16)(gb))
        t_need = k_rem2 - gt

        def ib(t, ipfx):
            cand = ipfx + (jnp.int32(1) << (12 - t))

            def cb2(i, acc):
                s = ckeys2[pl.ds(i, L)]
                iv = cidx2[pl.ds(i, L)]
                return acc + jnp.where((s == big_t) & (iv < cand), 1, 0)

            c = jnp.sum(plsc.parallel_loop(0, nv2 * L, L, carry=zeros16)(cb2))
            return jnp.where(c < t_need, cand, ipfx)

        cstar = lax.fori_loop(0, 13, ib, jnp.int32(0))

        @plsc.parallel_loop(0, N, L, unroll=8)
        def ob(i):
            s = keys[pl.ds(i, L)]
            iv = lane + i
            m = (s > big_t) | ((s == big_t) & (iv <= cstar))
            orow[pl.ds(i, L)] = jnp.where(m, 1.0, 0.0)

        pltpu.sync_copy(orow, out_hbm.at[row])


def kernel(score):
    return _topk_mask(score)


# speedup vs baseline: 9.4357x; 1.1037x over previous
import functools

import jax
import jax.numpy as jnp
from jax import lax
from jax.experimental import pallas as pl
from jax.experimental.pallas import tpu as pltpu
from jax.experimental.pallas import tpu_sc as plsc

B = 64
N = 8192
K = 256
L = 16
NBINS = 2048
CBINS = 128
ROWS_PER_W = 2
_INT_MIN = -(2**31)

_mesh = plsc.VectorSubcoreMesh(core_axis_name="c", subcore_axis_name="s")


@functools.partial(
    pl.kernel,
    mesh=_mesh,
    compiler_params=pltpu.CompilerParams(needs_layout_passes=False),
    out_type=jax.ShapeDtypeStruct((B, N), jnp.float32),
    scratch_types=[
        pltpu.VMEM((N,), jnp.float32),
        pltpu.VMEM((N,), jnp.int32),
        pltpu.VMEM((N,), jnp.float32),
        pltpu.VMEM((NBINS,), jnp.int32),
        pltpu.VMEM((CBINS,), jnp.int32),
        pltpu.VMEM((N + 2 * L,), jnp.int32),
        pltpu.VMEM((N + 2 * L,), jnp.int32),
        pltpu.VMEM((N + 2 * L,), jnp.int32),
        pltpu.VMEM((N + 2 * L,), jnp.int32),
    ],
)
def _topk_mask(
    score_hbm, out_hbm, rowf, keys, orow, hist, hist2, ckeys, cidx,
    ckeys2, cidx2,
):
    wid = lax.axis_index("s") * 2 + lax.axis_index("c")
    lane = lax.iota(jnp.int32, L)
    zeros16 = jnp.zeros((L,), jnp.int32)
    ones16 = jnp.ones((L,), jnp.int32)

    def walk(href, nbins, kk):
        def wcond(c):
            return c[1] == 0

        def wbody(c):
            i, found, j0, above, _cs, _rv = c
            v = href[pl.ds(nbins - L * (i + 1), L)]
            rv = lax.rev(v, (0,))
            cs = above + plsc.cumsum(rv)
            crossed = cs >= kk
            pc = plsc.all_reduce_population_count(crossed)
            pc0 = pc[0] if pc.ndim else pc
            jf = plsc.all_reduce_ffs(crossed)
            j = jf[0] if jf.ndim else jf
            hit = pc0 > 0
            return (
                i + 1,
                jnp.where(hit, 1, 0),
                jnp.where(hit, j, j0),
                jnp.where(hit, above, cs[L - 1]),
                jnp.where(hit, cs, _cs),
                jnp.where(hit, rv, _rv),
            )

        wi, _, wj, _, wcs, wrv = lax.while_loop(
            wcond, wbody, (0, 0, 0, 0, zeros16, zeros16)
        )
        bin_ = nbins - 1 - L * (wi - 1) - wj
        above = jnp.sum(jnp.where(lane == wj, wcs - wrv, 0))
        return bin_, above

    for r in range(ROWS_PER_W):
        row = wid * ROWS_PER_W + r
        pltpu.sync_copy(score_hbm.at[row], rowf)

        @plsc.parallel_loop(0, NBINS, L, unroll=8)
        def zh(i):
            hist[pl.ds(i, L)] = zeros16

        @plsc.parallel_loop(0, CBINS, L, unroll=8)
        def zc(i):
            hist2[pl.ds(i, L)] = zeros16

        @plsc.parallel_loop(0, N, L, unroll=8)
        def p1(i):
            f = rowf[pl.ds(i, L)]
            bits = lax.bitcast_convert_type(f, jnp.int32)
            s = jnp.where(bits >= 0, bits, _INT_MIN - bits)
            keys[pl.ds(i, L)] = s
            plsc.addupdate_scatter(hist, [(s >> 21) + 1024], ones16)

        bb, above = walk(hist, NBINS, K)
        k_rem = K - above

        def cp(i, off):
            s = keys[pl.ds(i, L)]
            m = ((s >> 21) + 1024) == bb
            plsc.store_compressed(ckeys.at[pl.ds(off, L)], s, mask=m)
            plsc.store_compressed(cidx.at[pl.ds(off, L)], lane + i, mask=m)
            pcv = plsc.all_reduce_population_count(m)
            return off + (pcv[0] if pcv.ndim else pcv)

        n = plsc.parallel_loop(0, N, L, unroll=4, carry=jnp.int32(0))(cp)
        ckeys[pl.ds(n, L)] = jnp.full((L,), _INT_MIN, jnp.int32)
        nv = (n + L - 1) // L

        def h2(i):
            s = ckeys[pl.ds(i, L)]
            plsc.addupdate_scatter(
                hist2, [(s >> 14) & 127], ones16, mask=s != _INT_MIN
            )

        plsc.parallel_loop(0, nv * L, L)(h2)
        sb, above2 = walk(hist2, CBINS, k_rem)
        k_rem2 = k_rem - above2

        def cp2(i, off):
            s = ckeys[pl.ds(i, L)]
            iv = cidx[pl.ds(i, L)]
            m = (((s >> 14) & 127) == sb) & (s != _INT_MIN)
            plsc.store_compressed(ckeys2.at[pl.ds(off, L)], s, mask=m)
            plsc.store_compressed(cidx2.at[pl.ds(off, L)], iv, mask=m)
            pcv = plsc.all_reduce_population_count(m)
            return off + (pcv[0] if pcv.ndim else pcv)

        n2 = plsc.parallel_loop(0, nv * L, L, carry=jnp.int32(0))(cp2)
        ckeys2[pl.ds(n2, L)] = jnp.full((L,), _INT_MIN, jnp.int32)
        nv2 = (n2 + L - 1) // L

        def rb(t, thr):
            cand = thr | (jnp.int32(1) << (13 - t))

            def cb_(i, acc):
                return acc + jnp.where(ckeys2[pl.ds(i, L)] >= cand, 1, 0)

            c = jnp.sum(plsc.parallel_loop(0, nv2 * L, L, carry=zeros16)(cb_))
            return jnp.where(c >= k_rem2, cand, thr)

        base2 = ((bb - 1024) << 21) | (sb << 14)
        big_t = lax.fori_loop(0, 14, rb, base2)

        def gb(i, acc):
            return acc + jnp.where(ckeys2[pl.ds(i, L)] > big_t, 1, 0)

        gt = jnp.sum(plsc.parallel_loop(0, nv2 * L, L, carry=zeros16)(gb))
        t_need = k_rem2 - gt

        def ib(t, ipfx):
            cand = ipfx + (jnp.int32(1) << (12 - t))

            def cb2(i, acc):
                s = ckeys2[pl.ds(i, L)]
                iv = cidx2[pl.ds(i, L)]
                return acc + jnp.where((s == big_t) & (iv < cand), 1, 0)

            c = jnp.sum(plsc.parallel_loop(0, nv2 * L, L, carry=zeros16)(cb2))
            return jnp.where(c < t_need, cand, ipfx)

        cstar = lax.fori_loop(0, 13, ib, jnp.int32(0))

        @plsc.parallel_loop(0, N, L, unroll=8)
        def ob(i):
            s = keys[pl.ds(i, L)]
            iv = lane + i
            m = (s > big_t) | ((s == big_t) & (iv <= cstar))
            orow[pl.ds(i, L)] = jnp.where(m, 1.0, 0.0)

        pltpu.sync_copy(orow, out_hbm.at[row])


def kernel(score):
    return _topk_mask(score)


# speedup vs baseline: 10.1345x; 1.0741x over previous
import functools

import jax
import jax.numpy as jnp
from jax import lax
from jax.experimental import pallas as pl
from jax.experimental.pallas import tpu as pltpu
from jax.experimental.pallas import tpu_sc as plsc

B = 64
N = 8192
K = 256
L = 16
NBINS = 2048
CBINS = 128
ROWS_PER_W = 2
_INT_MIN = -(2**31)

_mesh = plsc.VectorSubcoreMesh(core_axis_name="c", subcore_axis_name="s")


@functools.partial(
    pl.kernel,
    mesh=_mesh,
    compiler_params=pltpu.CompilerParams(needs_layout_passes=False),
    out_type=jax.ShapeDtypeStruct((B, N), jnp.float32),
    scratch_types=[
        pltpu.VMEM((N,), jnp.float32),
        pltpu.VMEM((N,), jnp.float32),
        pltpu.VMEM((N,), jnp.int32),
        pltpu.VMEM((N,), jnp.float32),
        pltpu.VMEM((N,), jnp.float32),
        pltpu.VMEM((NBINS,), jnp.int32),
        pltpu.VMEM((CBINS,), jnp.int32),
        pltpu.VMEM((N + 2 * L,), jnp.int32),
        pltpu.VMEM((N + 2 * L,), jnp.int32),
        pltpu.VMEM((N + 2 * L,), jnp.int32),
        pltpu.VMEM((N + 2 * L,), jnp.int32),
        pltpu.SemaphoreType.DMA,
        pltpu.SemaphoreType.DMA,
        pltpu.SemaphoreType.DMA,
        pltpu.SemaphoreType.DMA,
    ],
)
def _topk_mask(
    score_hbm, out_hbm, rowf_a, rowf_b, keys, orow_a, orow_b, hist, hist2,
    ckeys, cidx, ckeys2, cidx2, sem_in_a, sem_in_b, sem_out_a, sem_out_b,
):
    wid = lax.axis_index("s") * 2 + lax.axis_index("c")
    lane = lax.iota(jnp.int32, L)
    zeros16 = jnp.zeros((L,), jnp.int32)
    ones16 = jnp.ones((L,), jnp.int32)

    def walk(href, nbins, kk):
        def wcond(c):
            return c[1] == 0

        def wbody(c):
            i, found, j0, above, _cs, _rv = c
            v = href[pl.ds(nbins - L * (i + 1), L)]
            rv = lax.rev(v, (0,))
            cs = above + plsc.cumsum(rv)
            crossed = cs >= kk
            pc = plsc.all_reduce_population_count(crossed)
            pc0 = pc[0] if pc.ndim else pc
            jf = plsc.all_reduce_ffs(crossed)
            j = jf[0] if jf.ndim else jf
            hit = pc0 > 0
            return (
                i + 1,
                jnp.where(hit, 1, 0),
                jnp.where(hit, j, j0),
                jnp.where(hit, above, cs[L - 1]),
                jnp.where(hit, cs, _cs),
                jnp.where(hit, rv, _rv),
            )

        wi, _, wj, _, wcs, wrv = lax.while_loop(
            wcond, wbody, (0, 0, 0, 0, zeros16, zeros16)
        )
        bin_ = nbins - 1 - L * (wi - 1) - wj
        above = jnp.sum(jnp.where(lane == wj, wcs - wrv, 0))
        return bin_, above

    rowfs = (rowf_a, rowf_b)
    orows = (orow_a, orow_b)
    in_sems = (sem_in_a, sem_in_b)
    out_sems = (sem_out_a, sem_out_b)
    in_cps = [
        pltpu.async_copy(score_hbm.at[wid * ROWS_PER_W + r], rowfs[r], in_sems[r])
        for r in range(ROWS_PER_W)
    ]
    out_cps = []

    for r in range(ROWS_PER_W):
        row = wid * ROWS_PER_W + r
        rowf = rowfs[r]
        orow = orows[r]
        in_cps[r].wait()

        @plsc.parallel_loop(0, NBINS, L, unroll=8)
        def zh(i):
            hist[pl.ds(i, L)] = zeros16

        @plsc.parallel_loop(0, CBINS, L, unroll=8)
        def zc(i):
            hist2[pl.ds(i, L)] = zeros16

        @plsc.parallel_loop(0, N, L, unroll=8)
        def p1(i):
            f = rowf[pl.ds(i, L)]
            bits = lax.bitcast_convert_type(f, jnp.int32)
            s = jnp.where(bits >= 0, bits, _INT_MIN - bits)
            keys[pl.ds(i, L)] = s
            plsc.addupdate_scatter(hist, [(s >> 21) + 1024], ones16)

        bb, above = walk(hist, NBINS, K)
        k_rem = K - above

        def cp(i, off):
            s = keys[pl.ds(i, L)]
            m = ((s >> 21) + 1024) == bb
            plsc.store_compressed(ckeys.at[pl.ds(off, L)], s, mask=m)
            plsc.store_compressed(cidx.at[pl.ds(off, L)], lane + i, mask=m)
            pcv = plsc.all_reduce_population_count(m)
            return off + (pcv[0] if pcv.ndim else pcv)

        n = plsc.parallel_loop(0, N, L, unroll=4, carry=jnp.int32(0))(cp)
        ckeys[pl.ds(n, L)] = jnp.full((L,), _INT_MIN, jnp.int32)
        nv = (n + L - 1) // L

        def h2(i):
            s = ckeys[pl.ds(i, L)]
            plsc.addupdate_scatter(
                hist2, [(s >> 14) & 127], ones16, mask=s != _INT_MIN
            )

        plsc.parallel_loop(0, nv * L, L)(h2)
        sb, above2 = walk(hist2, CBINS, k_rem)
        k_rem2 = k_rem - above2

        def cp2(i, off):
            s = ckeys[pl.ds(i, L)]
            iv = cidx[pl.ds(i, L)]
            m = (((s >> 14) & 127) == sb) & (s != _INT_MIN)
            plsc.store_compressed(ckeys2.at[pl.ds(off, L)], s, mask=m)
            plsc.store_compressed(cidx2.at[pl.ds(off, L)], iv, mask=m)
            pcv = plsc.all_reduce_population_count(m)
            return off + (pcv[0] if pcv.ndim else pcv)

        n2 = plsc.parallel_loop(0, nv * L, L, carry=jnp.int32(0))(cp2)
        ckeys2[pl.ds(n2, L)] = jnp.full((L,), _INT_MIN, jnp.int32)
        nv2 = (n2 + L - 1) // L

        def rb(t, thr):
            cand = thr | (jnp.int32(1) << (13 - t))

            def cb_(i, acc):
                return acc + jnp.where(ckeys2[pl.ds(i, L)] >= cand, 1, 0)

            c = jnp.sum(plsc.parallel_loop(0, nv2 * L, L, carry=zeros16)(cb_))
            return jnp.where(c >= k_rem2, cand, thr)

        base2 = ((bb - 1024) << 21) | (sb << 14)
        big_t = lax.fori_loop(0, 14, rb, base2)

        def gb(i, acc):
            return acc + jnp.where(ckeys2[pl.ds(i, L)] > big_t, 1, 0)

        gt = jnp.sum(plsc.parallel_loop(0, nv2 * L, L, carry=zeros16)(gb))
        t_need = k_rem2 - gt

        def ib(t, ipfx):
            cand = ipfx + (jnp.int32(1) << (12 - t))

            def cb2(i, acc):
                s = ckeys2[pl.ds(i, L)]
                iv = cidx2[pl.ds(i, L)]
                return acc + jnp.where((s == big_t) & (iv < cand), 1, 0)

            c = jnp.sum(plsc.parallel_loop(0, nv2 * L, L, carry=zeros16)(cb2))
            return jnp.where(c < t_need, cand, ipfx)

        cstar = lax.fori_loop(0, 13, ib, jnp.int32(0))

        @plsc.parallel_loop(0, N, L, unroll=8)
        def ob(i):
            s = keys[pl.ds(i, L)]
            iv = lane + i
            m = (s > big_t) | ((s == big_t) & (iv <= cstar))
            orow[pl.ds(i, L)] = jnp.where(m, 1.0, 0.0)

        out_cps.append(pltpu.async_copy(orow, out_hbm.at[row], out_sems[r]))

    for cp in out_cps:
        cp.wait()


def kernel(score):
    return _topk_mask(score)


# speedup vs baseline: 10.1376x; 1.0003x over previous
import functools

import jax
import jax.numpy as jnp
from jax import lax
from jax.experimental import pallas as pl
from jax.experimental.pallas import tpu as pltpu
from jax.experimental.pallas import tpu_sc as plsc

B = 64
N = 8192
K = 256
L = 16
NBINS = 2048
CBINS = 128
ROWS_PER_W = 2
_INT_MIN = -(2**31)

_mesh = plsc.VectorSubcoreMesh(core_axis_name="c", subcore_axis_name="s")


@functools.partial(
    pl.kernel,
    mesh=_mesh,
    compiler_params=pltpu.CompilerParams(needs_layout_passes=False),
    out_type=jax.ShapeDtypeStruct((B, N), jnp.float32),
    scratch_types=[
        pltpu.VMEM((N,), jnp.float32),
        pltpu.VMEM((N,), jnp.float32),
        pltpu.VMEM((N,), jnp.int32),
        pltpu.VMEM((N,), jnp.float32),
        pltpu.VMEM((N,), jnp.float32),
        pltpu.VMEM((NBINS,), jnp.int32),
        pltpu.VMEM((CBINS,), jnp.int32),
        pltpu.VMEM((N + 2 * L,), jnp.int32),
        pltpu.VMEM((N + 2 * L,), jnp.int32),
        pltpu.VMEM((N + 2 * L,), jnp.int32),
        pltpu.VMEM((N + 2 * L,), jnp.int32),
        pltpu.SemaphoreType.DMA,
        pltpu.SemaphoreType.DMA,
        pltpu.SemaphoreType.DMA,
        pltpu.SemaphoreType.DMA,
    ],
)
def _topk_mask(
    score_hbm, out_hbm, rowf_a, rowf_b, keys, orow_a, orow_b, hist, hist2,
    ckeys, cidx, ckeys2, cidx2, sem_in_a, sem_in_b, sem_out_a, sem_out_b,
):
    wid = lax.axis_index("s") * 2 + lax.axis_index("c")
    lane = lax.iota(jnp.int32, L)
    zeros16 = jnp.zeros((L,), jnp.int32)
    ones16 = jnp.ones((L,), jnp.int32)

    def walk(href, nbins, kk, i0=0):
        def wcond(c):
            return c[1] == 0

        def wbody(c):
            i, found, j0, above, _cs, _rv = c
            v = href[pl.ds(nbins - L * (i + 1), L)]
            rv = lax.rev(v, (0,))
            cs = above + plsc.cumsum(rv)
            crossed = cs >= kk
            pc = plsc.all_reduce_population_count(crossed)
            pc0 = pc[0] if pc.ndim else pc
            jf = plsc.all_reduce_ffs(crossed)
            j = jf[0] if jf.ndim else jf
            hit = pc0 > 0
            return (
                i + 1,
                jnp.where(hit, 1, 0),
                jnp.where(hit, j, j0),
                jnp.where(hit, above, cs[L - 1]),
                jnp.where(hit, cs, _cs),
                jnp.where(hit, rv, _rv),
            )

        wi, _, wj, _, wcs, wrv = lax.while_loop(
            wcond, wbody, (jnp.int32(i0) + 0, 0, 0, 0, zeros16, zeros16)
        )
        bin_ = nbins - 1 - L * (wi - 1) - wj
        above = jnp.sum(jnp.where(lane == wj, wcs - wrv, 0))
        return bin_, above

    rowfs = (rowf_a, rowf_b)
    orows = (orow_a, orow_b)
    in_sems = (sem_in_a, sem_in_b)
    out_sems = (sem_out_a, sem_out_b)
    in_cps = [
        pltpu.async_copy(score_hbm.at[wid * ROWS_PER_W + r], rowfs[r], in_sems[r])
        for r in range(ROWS_PER_W)
    ]
    out_cps = []

    for r in range(ROWS_PER_W):
        row = wid * ROWS_PER_W + r
        rowf = rowfs[r]
        orow = orows[r]
        in_cps[r].wait()

        @plsc.parallel_loop(0, NBINS, L, unroll=8)
        def zh(i):
            hist[pl.ds(i, L)] = zeros16

        @plsc.parallel_loop(0, CBINS, L, unroll=8)
        def zc(i):
            hist2[pl.ds(i, L)] = zeros16

        def p1(i, mx):
            f = rowf[pl.ds(i, L)]
            bits = lax.bitcast_convert_type(f, jnp.int32)
            s = jnp.where(bits >= 0, bits, _INT_MIN - bits)
            keys[pl.ds(i, L)] = s
            plsc.addupdate_scatter(hist, [(s >> 21) + 1024], ones16)
            return jnp.maximum(mx, s)

        minv16 = jnp.full((L,), _INT_MIN, jnp.int32)
        mxv = plsc.parallel_loop(0, N, L, unroll=8, carry=minv16)(p1)
        max_bin = (jnp.max(mxv) >> 21) + 1024

        bb, above = walk(hist, NBINS, K, i0=(NBINS - 1 - max_bin) // L)
        k_rem = K - above

        def cp(i, off):
            s = keys[pl.ds(i, L)]
            m = ((s >> 21) + 1024) == bb
            plsc.store_compressed(ckeys.at[pl.ds(off, L)], s, mask=m)
            plsc.store_compressed(cidx.at[pl.ds(off, L)], lane + i, mask=m)
            pcv = plsc.all_reduce_population_count(m)
            return off + (pcv[0] if pcv.ndim else pcv)

        n = plsc.parallel_loop(0, N, L, unroll=4, carry=jnp.int32(0))(cp)
        ckeys[pl.ds(n, L)] = jnp.full((L,), _INT_MIN, jnp.int32)
        nv = (n + L - 1) // L

        def h2(i):
            s = ckeys[pl.ds(i, L)]
            plsc.addupdate_scatter(
                hist2, [(s >> 14) & 127], ones16, mask=s != _INT_MIN
            )

        plsc.parallel_loop(0, nv * L, L)(h2)
        sb, above2 = walk(hist2, CBINS, k_rem)
        k_rem2 = k_rem - above2

        def cp2(i, off):
            s = ckeys[pl.ds(i, L)]
            iv = cidx[pl.ds(i, L)]
            m = (((s >> 14) & 127) == sb) & (s != _INT_MIN)
            plsc.store_compressed(ckeys2.at[pl.ds(off, L)], s, mask=m)
            plsc.store_compressed(cidx2.at[pl.ds(off, L)], iv, mask=m)
            pcv = plsc.all_reduce_population_count(m)
            return off + (pcv[0] if pcv.ndim else pcv)

        n2 = plsc.parallel_loop(0, nv * L, L, carry=jnp.int32(0))(cp2)
        ckeys2[pl.ds(n2, L)] = jnp.full((L,), _INT_MIN, jnp.int32)
        nv2 = (n2 + L - 1) // L

        def rb(t, thr):
            cand = thr | (jnp.int32(1) << (13 - t))

            def cb_(i, acc):
                return acc + jnp.where(ckeys2[pl.ds(i, L)] >= cand, 1, 0)

            c = jnp.sum(plsc.parallel_loop(0, nv2 * L, L, carry=zeros16)(cb_))
            return jnp.where(c >= k_rem2, cand, thr)

        base2 = ((bb - 1024) << 21) | (sb << 14)
        big_t = lax.fori_loop(0, 14, rb, base2)

        def gb(i, acc):
            return acc + jnp.where(ckeys2[pl.ds(i, L)] > big_t, 1, 0)

        gt = jnp.sum(plsc.parallel_loop(0, nv2 * L, L, carry=zeros16)(gb))
        t_need = k_rem2 - gt

        def ib(t, ipfx):
            cand = ipfx + (jnp.int32(1) << (12 - t))

            def cb2(i, acc):
                s = ckeys2[pl.ds(i, L)]
                iv = cidx2[pl.ds(i, L)]
                return acc + jnp.where((s == big_t) & (iv < cand), 1, 0)

            c = jnp.sum(plsc.parallel_loop(0, nv2 * L, L, carry=zeros16)(cb2))
            return jnp.where(c < t_need, cand, ipfx)

        cstar = lax.fori_loop(0, 13, ib, jnp.int32(0))

        @plsc.parallel_loop(0, N, L, unroll=8)
        def ob(i):
            s = keys[pl.ds(i, L)]
            iv = lane + i
            m = (s > big_t) | ((s == big_t) & (iv <= cstar))
            orow[pl.ds(i, L)] = jnp.where(m, 1.0, 0.0)

        out_cps.append(pltpu.async_copy(orow, out_hbm.at[row], out_sems[r]))

    for cp in out_cps:
        cp.wait()


def kernel(score):
    return _topk_mask(score)
